# baseline (speedup 1.0000x reference)
import functools

import jax
import jax.numpy as jnp
from jax import lax
from jax.experimental import pallas as pl
from jax.experimental.pallas import tpu as pltpu
from jax.experimental.pallas import tpu_sc as plsc

N = 10000
E = 320000
G = 16
FX, FE, FU, H = 128, 16, 32, 64
NN, EE = 2 * N, 2 * E

N_BLK = 2000
E_BLK = 8000
NBN = N // N_BLK
NBE = E // E_BLK

_NC, _NS = 2, 16
_NW = _NC * _NS
_EW = EE // _NW
_GK = 400
_GCH = _EW // _GK
_SK = 128
_SCH = _EW // _SK
_STAIL = _EW - _SCH * _SK
_PR = 1248
_NT = NN - _NS * _PR


def _full(shape):
    nd = len(shape)
    return pl.BlockSpec(shape, lambda g, i: (0,) * nd)


def _wspec(shape):
    nd = len(shape)
    return pl.BlockSpec((1,) + shape, lambda g, i: (g,) + (0,) * nd)


def _sc_gather_body(a_hbm, b_hbm, src_hbm, dst_hbm, gab_hbm,
                    ia0, ia1, ib0, ib1, ra0, ra1, rb0, rb1,
                    gsa0, gsa1, gsb0, gsb1, wsa0, wsa1, wsb0, wsb1):
    ia, ib = (ia0, ia1), (ib0, ib1)
    ra, rb = (ra0, ra1), (rb0, rb1)
    gsa, gsb = (gsa0, gsa1), (gsb0, gsb1)
    wsa, wsb = (wsa0, wsa1), (wsb0, wsb1)
    wid = lax.axis_index("s") * _NC + lax.axis_index("c")
    base0 = wid * _EW

    def load_and_gather(c, bi):
        off = base0 + c * _GK
        pltpu.sync_copy(src_hbm.at[pl.ds(off, _GK)], ia[bi])
        pltpu.sync_copy(dst_hbm.at[pl.ds(off, _GK)], ib[bi])
        ha = pltpu.async_copy(a_hbm.at[ia[bi]], ra[bi], gsa[bi])
        hb = pltpu.async_copy(b_hbm.at[ib[bi]], rb[bi], gsb[bi])
        return ha, hb

    gh = {0: load_and_gather(0, 0)}
    wh = {}
    for c in range(_GCH):
        bi = c & 1
        ha, hb = gh[c]
        ha.wait()
        hb.wait()
        off = base0 + c * _GK
        wh[c] = (
            pltpu.async_copy(ra[bi], gab_hbm.at[pl.ds(off, _GK), pl.ds(0, H)],
                             wsa[bi]),
            pltpu.async_copy(rb[bi], gab_hbm.at[pl.ds(off, _GK), pl.ds(H, H)],
                             wsb[bi]),
        )
        if c + 1 < _GCH:
            if c >= 1:
                for hnd in wh[c - 1]:
                    hnd.wait()
            gh[c + 1] = load_and_gather(c + 1, bi ^ 1)
    for c in range(max(_GCH - 2, 0), _GCH):
        for hnd in wh[c]:
            hnd.wait()


@functools.partial(
    pl.kernel,
    out_type=jax.ShapeDtypeStruct((EE, 2 * H), jnp.bfloat16),
    mesh=plsc.VectorSubcoreMesh(core_axis_name="c", subcore_axis_name="s"),
    scratch_types=([pltpu.VMEM((_GK,), jnp.int32)] * 4
                   + [pltpu.VMEM((_GK, H), jnp.bfloat16)] * 4
                   + [pltpu.SemaphoreType.DMA] * 8),
    compiler_params=pltpu.CompilerParams(use_tc_tiling_on_sc=False),
)
def _sc_gather(a_hbm, b_hbm, src_hbm, dst_hbm, gab_hbm, *scr):
    _sc_gather_body(a_hbm, b_hbm, src_hbm, dst_hbm, gab_hbm, *scr)


def _sc_scatter_impl(with_vals, idx_hbm, vals_hbm, out_hbm, acc, zbuf,
                     iv0, iv1, rv0, rv1, it, rt, s0, s1, s2, s3):
    iv, rv = (iv0, iv1), (rv0, rv1)
    isem, vsem = (s0, s1), (s2, s3)
    cid = lax.axis_index("c")
    sid = lax.axis_index("s")
    wid = sid * _NC + cid
    base0 = wid * _EW

    def zb(i, _):
        zbuf[i, :] = jnp.zeros((16,), jnp.float32)
        return 0
    lax.fori_loop(0, _PR, zb, 0)
    pltpu.sync_copy(zbuf, acc.at[pl.ds(sid * _PR, _PR)])

    @pl.when(sid == _NS - 1)
    def _():
        pltpu.sync_copy(zbuf.at[pl.ds(0, _NT)], acc.at[pl.ds(_NS * _PR, _NT)])

    if not with_vals:
        def od(i, _):
            rv0[i, :] = jnp.ones((16,), jnp.float32)
            return 0
        lax.fori_loop(0, _SK, od, 0)

        def ot(i, _):
            rt[i, :] = jnp.ones((16,), jnp.float32)
            return 0
        lax.fori_loop(0, _STAIL, ot, 0)
    plsc.subcore_barrier()

    def load(c, bi):
        off = base0 + c * _SK
        h_i = pltpu.async_copy(idx_hbm.at[pl.ds(off, _SK)], iv[bi], isem[bi])
        h_v = None
        if with_vals:
            h_v = pltpu.async_copy(vals_hbm.at[pl.ds(off, _SK)], rv[bi],
                                   vsem[bi])
        return h_i, h_v

    lh = {0: load(0, 0)}
    for c in range(_SCH):
        bi = c & 1
        h_i, h_v = lh[c]
        h_i.wait()
        if h_v is not None:
            h_v.wait()
        if c + 1 < _SCH:
            lh[c + 1] = load(c + 1, bi ^ 1)
        src_rows = rv[bi] if with_vals else rv0
        pltpu.sync_copy(src_rows, acc.at[iv[bi]], add=True)
    off = base0 + _SCH * _SK
    pltpu.sync_copy(idx_hbm.at[pl.ds(off, _STAIL)], it)
    if with_vals:
        pltpu.sync_copy(vals_hbm.at[pl.ds(off, _STAIL)], rt)
    pltpu.sync_copy(rt, acc.at[it], add=True)

    plsc.subcore_barrier()
    row0 = sid * _PR
    pltpu.sync_copy(acc.at[pl.ds(row0, _PR)], zbuf)
    pltpu.sync_copy(zbuf, out_hbm.at[cid, pl.ds(row0, _PR)])

    @pl.when(sid == _NS - 1)
    def _():
        pltpu.sync_copy(acc.at[pl.ds(_NS * _PR, _NT)], rt)
        pltpu.sync_copy(rt, out_hbm.at[cid, pl.ds(_NS * _PR, _NT)])


def _make_sc_scatter(with_vals):
    scratch = ([pltpu.VMEM_SHARED((NN, FE), jnp.float32),
                pltpu.VMEM((_PR, FE), jnp.float32)]
               + [pltpu.VMEM((_SK,), jnp.int32)] * 2
               + [pltpu.VMEM((_SK, FE), jnp.float32)] * 2
               + [pltpu.VMEM((_STAIL,), jnp.int32),
                  pltpu.VMEM((_STAIL, FE), jnp.float32)]
               + [pltpu.SemaphoreType.DMA] * 4)
    mesh = plsc.VectorSubcoreMesh(core_axis_name="c", subcore_axis_name="s")
    out_type = jax.ShapeDtypeStruct((_NC, NN, FE), jnp.float32)
    cp = pltpu.CompilerParams(use_tc_tiling_on_sc=False)
    if with_vals:
        @functools.partial(pl.kernel, out_type=out_type, mesh=mesh,
                           scratch_types=scratch, compiler_params=cp)
        def k(idx_hbm, vals_hbm, out_hbm, *scr):
            _sc_scatter_impl(True, idx_hbm, vals_hbm, out_hbm, *scr)
    else:
        @functools.partial(pl.kernel, out_type=out_type, mesh=mesh,
                           scratch_types=scratch, compiler_params=cp)
        def k(idx_hbm, out_hbm, *scr):
            _sc_scatter_impl(False, idx_hbm, None, out_hbm, *scr)
    return k


_sc_scatter_vals = _make_sc_scatter(True)
_sc_scatter_ones = _make_sc_scatter(False)


def _prep_body(x_ref, batch_ref, u_ref, wsrc_ref, wdst_ref, wu_ref, b1_ref,
               a_ref, b_ref):
    ue = u_ref[0] @ wu_ref[0] + b1_ref[0]
    oh = (batch_ref[...] ==
          lax.broadcasted_iota(jnp.int32, (N_BLK, G), 1)).astype(jnp.float32)
    a_ref[...] = (x_ref[...] @ wsrc_ref[0] + oh @ ue).astype(jnp.bfloat16)
    b_ref[...] = (x_ref[...] @ wdst_ref[0]).astype(jnp.bfloat16)


def _prep(x, batch2d, u, wsrc, wdst, wu, b1):
    return pl.pallas_call(
        _prep_body,
        grid=(2, NBN),
        in_specs=[
            pl.BlockSpec((N_BLK, FX), lambda g, i: (g * NBN + i, 0)),
            pl.BlockSpec((N_BLK, 1), lambda g, i: (g * NBN + i, 0)),
            _wspec((G, FU)), _wspec((FX, H)), _wspec((FX, H)),
            _wspec((FU, H)), _wspec((1, H)),
        ],
        out_specs=[
            pl.BlockSpec((N_BLK, H), lambda g, i: (g * NBN + i, 0)),
            pl.BlockSpec((N_BLK, H), lambda g, i: (g * NBN + i, 0)),
        ],
        out_shape=[
            jax.ShapeDtypeStruct((NN, H), jnp.bfloat16),
            jax.ShapeDtypeStruct((NN, H), jnp.bfloat16),
        ],
    )(x, batch2d, u, wsrc, wdst, wu, b1)


def _edge_body(gab_ref, e_ref, we_ref, w2_ref, b2_ref, out_ref):
    g = (gab_ref[:, :H].astype(jnp.float32)
         + gab_ref[:, H:].astype(jnp.float32))
    h = jnp.maximum(g + e_ref[...] @ we_ref[0], 0.0)
    out_ref[...] = h @ w2_ref[0] + b2_ref[0]


def _edge(gab, e, we, w2, b2):
    return pl.pallas_call(
        _edge_body,
        grid=(2, NBE),
        in_specs=[
            pl.BlockSpec((E_BLK, 2 * H), lambda g, i: (g * NBE + i, 0)),
            pl.BlockSpec((E_BLK, FE), lambda g, i: (g * NBE + i, 0)),
            _wspec((FE, H)), _wspec((H, FE)), _wspec((1, FE)),
        ],
        out_specs=pl.BlockSpec((E_BLK, FE), lambda g, i: (g * NBE + i, 0)),
        out_shape=jax.ShapeDtypeStruct((EE, FE), jnp.float32),
    )(gab, e, we, w2, b2)


def _node_body(x_ref, s0_ref, s1_ref, c0_ref, c1c_ref, batch_ref, u_ref,
               vx_ref, va_ref, vu_ref, c1_ref, v2_ref, c2_ref,
               gwu_ref, gwx_ref, g1_ref, g2_ref, g2b_ref,
               xnew_ref, unew_ref, scr_ref):
    i = pl.program_id(1)

    @pl.when(i == 0)
    def _():
        scr_ref[...] = jnp.zeros_like(scr_ref)

    agg = ((s0_ref[...] + s1_ref[...])
           / jnp.maximum(c0_ref[...] + c1c_ref[...], 1.0))
    uu = u_ref[0] @ vu_ref[0] + c1_ref[0]
    oh = (batch_ref[...] ==
          lax.broadcasted_iota(jnp.int32, (N_BLK, G), 1)).astype(jnp.float32)
    hx = jnp.maximum(x_ref[...] @ vx_ref[0] + agg @ va_ref[0] + oh @ uu, 0.0)
    xn = hx @ v2_ref[0] + c2_ref[0]
    xnew_ref[...] = xn
    ones = jnp.ones((N_BLK, FX), jnp.float32)
    scr_ref[...] += oh.T @ jnp.concatenate([xn, ones], axis=1)

    @pl.when(i == NBN - 1)
    def _():
        xagg = scr_ref[:, :FX] / jnp.maximum(scr_ref[:, FX:], 1.0)
        gin = jnp.maximum(
            u_ref[0] @ gwu_ref[0] + xagg @ gwx_ref[0] + g1_ref[0], 0.0)
        unew_ref[0] = gin @ g2_ref[0] + g2b_ref[0]


def _node(x, s0, s1, c0, c1c, batch2d, u, w):
    nspec = lambda width: pl.BlockSpec((N_BLK, width),
                                       lambda g, i: (g * NBN + i, 0))
    return pl.pallas_call(
        _node_body,
        grid=(2, NBN),
        in_specs=[
            nspec(FX), nspec(FE), nspec(FE), nspec(FE), nspec(FE), nspec(1),
            _wspec((G, FU)),
            _wspec((FX, H)), _wspec((FE, H)), _wspec((FU, H)), _wspec((1, H)),
            _wspec((H, FX)), _wspec((1, FX)),
            _wspec((FU, H)), _wspec((FX, H)), _wspec((1, H)), _wspec((H, FU)),
            _wspec((1, FU)),
        ],
        out_specs=[
            nspec(FX),
            pl.BlockSpec((1, G, FU), lambda g, i: (g, 0, 0)),
        ],
        out_shape=[
            jax.ShapeDtypeStruct((NN, FX), jnp.float32),
            jax.ShapeDtypeStruct((2, G, FU), jnp.float32),
        ],
        scratch_shapes=[pltpu.VMEM((G, 2 * FX), jnp.float32)],
    )(x, s0, s1, c0, c1c, batch2d, u, w['vx'], w['va'], w['vu'], w['c1'],
      w['v2'], w['c2'], w['gwu'], w['gwx'], w['g1'], w['g2'], w['g2b'])


def _out_body(u1_ref, u2_ref, ow1_ref, ow2_ref, o1_ref, o2_ref, o2b_ref,
              out_ref):
    h = jnp.maximum(
        u1_ref[...] @ ow1_ref[...] + u2_ref[...] @ ow2_ref[...] + o1_ref[...],
        0.0)
    out_ref[...] = h @ o2_ref[...] + o2b_ref[...]


def _out(u1s, u2s, ow1, ow2, o1, o2, o2b):
    k = u1s.shape[0]
    f = lambda shape: pl.BlockSpec(shape, lambda i: (0,) * len(shape))
    return pl.pallas_call(
        _out_body,
        grid=(1,),
        in_specs=[f((k, FU)), f((k, FU)), f((FU, H)),
                  f((FU, H)), f((1, H)), f((H, 2)), f((1, 2))],
        out_specs=f((k, 2)),
        out_shape=jax.ShapeDtypeStruct((k, 2), jnp.float32),
    )(u1s, u2s, ow1, ow2, o1, o2, o2b)


def _stack_params(p1, p2):
    def s2(a, b):
        return jnp.stack([a, b])

    def split(p):
        w1, b1, w2, b2 = p['edge']
        ew = dict(wsrc=w1[:FX], wdst=w1[FX:2 * FX], we=w1[2 * FX:2 * FX + FE],
                  wu=w1[2 * FX + FE:], b1=b1.reshape(1, H), w2=w2,
                  b2=b2.reshape(1, FE))
        v1, c1, v2, c2 = p['node']
        nw = dict(vx=v1[:FX], va=v1[FX:FX + FE], vu=v1[FX + FE:],
                  c1=c1.reshape(1, H), v2=v2, c2=c2.reshape(1, FX),)
        g1w, g1b, g2w, g2b = p['glob']
        nw.update(gwu=g1w[:FU], gwx=g1w[FU:], g1=g1b.reshape(1, H), g2=g2w,
                  g2b=g2b.reshape(1, FU))
        return ew, nw

    ew1, nw1 = split(p1)
    ew2, nw2 = split(p2)
    ew = {k: s2(ew1[k], ew2[k]) for k in ew1}
    nw = {k: s2(nw1[k], nw2[k]) for k in nw1}
    return ew, nw


def kernel(x1, edge_index1, e1, u1, batch1, x2, edge_index2, e2, u2, batch2,
           params):
    n_iters = 2
    ew, nw = _stack_params(params['gnn1'], params['gnn2'])
    x = jnp.concatenate([x1, x2], axis=0)
    e = jnp.concatenate([e1, e2], axis=0)
    u = jnp.stack([u1, u2])
    batch2d = jnp.concatenate([batch1, batch2], axis=0).reshape(NN, 1)
    src = jnp.concatenate([edge_index1[0], edge_index2[0] + N], axis=0)
    dst = jnp.concatenate([edge_index1[1], edge_index2[1] + N], axis=0)

    cparts = _sc_scatter_ones(dst)
    us = []
    for it in range(n_iters):
        a, b = _prep(x, batch2d, u, ew['wsrc'], ew['wdst'], ew['wu'],
                     ew['b1'])
        gab = _sc_gather(a, b, src, dst)
        e = _edge(gab, e, ew['we'], ew['w2'], ew['b2'])
        sp = _sc_scatter_vals(dst, e)
        x, u = _node(x, sp[0], sp[1], cparts[0], cparts[1], batch2d, u, nw)
        us.append(u)

    ow, o1, o2, o2b = params['out']
    u1s = jnp.concatenate([us[0][0], us[1][0]], axis=0)
    u2s = jnp.concatenate([us[0][1], us[1][1]], axis=0)
    outs = _out(u1s, u2s, ow[:FU], ow[FU:], o1.reshape(1, H), o2,
                o2b.reshape(1, 2))
    return outs.reshape(n_iters, G, 2)


# speedup vs baseline: 1.0856x; 1.0856x over previous
import functools

import jax
import jax.numpy as jnp
from jax import lax
from jax.experimental import pallas as pl
from jax.experimental.pallas import tpu as pltpu
from jax.experimental.pallas import tpu_sc as plsc

N = 10000
E = 320000
G = 16
FX, FE, FU, H = 128, 16, 32, 64

N_BLK = 2000
E_BLK = 8000

_NC, _NS = 2, 16
_NW = _NC * _NS
_EW = E // _NW
_GK = 400
_GCH = _EW // _GK
_SK = 128
_SCH = _EW // _SK
_STAIL = _EW - _SCH * _SK
_PR = 624
_NT = N - _NS * _PR


def _full(shape):
    nd = len(shape)
    return pl.BlockSpec(shape, lambda i: (0,) * nd)


def _sc_gather_body(a_hbm, b_hbm, src_hbm, dst_hbm, gab_hbm,
                    ia0, ia1, ib0, ib1, ra0, ra1, rb0, rb1,
                    gsa0, gsa1, gsb0, gsb1, wsa0, wsa1, wsb0, wsb1):
    ia, ib = (ia0, ia1), (ib0, ib1)
    ra, rb = (ra0, ra1), (rb0, rb1)
    gsa, gsb = (gsa0, gsa1), (gsb0, gsb1)
    wsa, wsb = (wsa0, wsa1), (wsb0, wsb1)
    wid = lax.axis_index("s") * _NC + lax.axis_index("c")
    base0 = wid * _EW

    def load_and_gather(c, bi):
        off = base0 + c * _GK
        pltpu.sync_copy(src_hbm.at[pl.ds(off, _GK)], ia[bi])
        pltpu.sync_copy(dst_hbm.at[pl.ds(off, _GK)], ib[bi])
        ha = pltpu.async_copy(a_hbm.at[ia[bi]], ra[bi], gsa[bi])
        hb = pltpu.async_copy(b_hbm.at[ib[bi]], rb[bi], gsb[bi])
        return ha, hb

    gh = {0: load_and_gather(0, 0)}
    wh = {}
    for c in range(_GCH):
        bi = c & 1
        ha, hb = gh[c]
        ha.wait()
        hb.wait()
        off = base0 + c * _GK
        wh[c] = (
            pltpu.async_copy(ra[bi], gab_hbm.at[pl.ds(off, _GK), pl.ds(0, H)],
                             wsa[bi]),
            pltpu.async_copy(rb[bi], gab_hbm.at[pl.ds(off, _GK), pl.ds(H, H)],
                             wsb[bi]),
        )
        if c + 1 < _GCH:
            if c >= 1:
                for hnd in wh[c - 1]:
                    hnd.wait()
            gh[c + 1] = load_and_gather(c + 1, bi ^ 1)
    for c in range(max(_GCH - 2, 0), _GCH):
        for hnd in wh[c]:
            hnd.wait()


@functools.partial(
    pl.kernel,
    out_type=jax.ShapeDtypeStruct((E, 2 * H), jnp.bfloat16),
    mesh=plsc.VectorSubcoreMesh(core_axis_name="c", subcore_axis_name="s"),
    scratch_types=([pltpu.VMEM((_GK,), jnp.int32)] * 4
                   + [pltpu.VMEM((_GK, H), jnp.bfloat16)] * 4
                   + [pltpu.SemaphoreType.DMA] * 8),
    compiler_params=pltpu.CompilerParams(use_tc_tiling_on_sc=False),
)
def _sc_gather(a_hbm, b_hbm, src_hbm, dst_hbm, gab_hbm, *scr):
    _sc_gather_body(a_hbm, b_hbm, src_hbm, dst_hbm, gab_hbm, *scr)


def _sc_scatter_impl(with_vals, idx_hbm, vals_hbm, out_hbm, acc, zbuf,
                     iv0, iv1, rv0, rv1, it, rt, s0, s1, s2, s3):
    iv, rv = (iv0, iv1), (rv0, rv1)
    isem, vsem = (s0, s1), (s2, s3)
    cid = lax.axis_index("c")
    sid = lax.axis_index("s")
    wid = sid * _NC + cid
    base0 = wid * _EW

    def zb(i, _):
        zbuf[i, :] = jnp.zeros((16,), jnp.float32)
        return 0
    lax.fori_loop(0, _PR, zb, 0)
    pltpu.sync_copy(zbuf, acc.at[pl.ds(sid * _PR, _PR)])

    @pl.when(sid == _NS - 1)
    def _():
        pltpu.sync_copy(zbuf.at[pl.ds(0, _NT)], acc.at[pl.ds(_NS * _PR, _NT)])

    if not with_vals:
        def od(i, _):
            rv0[i, :] = jnp.ones((16,), jnp.float32)
            return 0
        lax.fori_loop(0, _SK, od, 0)

        def ot(i, _):
            rt[i, :] = jnp.ones((16,), jnp.float32)
            return 0
        lax.fori_loop(0, _STAIL, ot, 0)
    plsc.subcore_barrier()

    def load(c, bi):
        off = base0 + c * _SK
        h_i = pltpu.async_copy(idx_hbm.at[pl.ds(off, _SK)], iv[bi], isem[bi])
        h_v = None
        if with_vals:
            h_v = pltpu.async_copy(vals_hbm.at[pl.ds(off, _SK)], rv[bi],
                                   vsem[bi])
        return h_i, h_v

    lh = {0: load(0, 0)}
    for c in range(_SCH):
        bi = c & 1
        h_i, h_v = lh[c]
        h_i.wait()
        if h_v is not None:
            h_v.wait()
        if c + 1 < _SCH:
            lh[c + 1] = load(c + 1, bi ^ 1)
        src_rows = rv[bi] if with_vals else rv0
        pltpu.sync_copy(src_rows, acc.at[iv[bi]], add=True)
    off = base0 + _SCH * _SK
    pltpu.sync_copy(idx_hbm.at[pl.ds(off, _STAIL)], it)
    if with_vals:
        pltpu.sync_copy(vals_hbm.at[pl.ds(off, _STAIL)], rt)
    pltpu.sync_copy(rt, acc.at[it], add=True)

    plsc.subcore_barrier()
    row0 = sid * _PR
    pltpu.sync_copy(acc.at[pl.ds(row0, _PR)], zbuf)
    pltpu.sync_copy(zbuf, out_hbm.at[cid, pl.ds(row0, _PR)])

    @pl.when(sid == _NS - 1)
    def _():
        pltpu.sync_copy(acc.at[pl.ds(_NS * _PR, _NT)], rt)
        pltpu.sync_copy(rt, out_hbm.at[cid, pl.ds(_NS * _PR, _NT)])


def _make_sc_scatter(with_vals):
    scratch = ([pltpu.VMEM_SHARED((N, FE), jnp.float32),
                pltpu.VMEM((_PR, FE), jnp.float32)]
               + [pltpu.VMEM((_SK,), jnp.int32)] * 2
               + [pltpu.VMEM((_SK, FE), jnp.float32)] * 2
               + [pltpu.VMEM((_STAIL,), jnp.int32),
                  pltpu.VMEM((_STAIL, FE), jnp.float32)]
               + [pltpu.SemaphoreType.DMA] * 4)
    mesh = plsc.VectorSubcoreMesh(core_axis_name="c", subcore_axis_name="s")
    out_type = jax.ShapeDtypeStruct((_NC, N, FE), jnp.float32)
    cp = pltpu.CompilerParams(use_tc_tiling_on_sc=False)
    if with_vals:
        @functools.partial(pl.kernel, out_type=out_type, mesh=mesh,
                           scratch_types=scratch, compiler_params=cp)
        def k(idx_hbm, vals_hbm, out_hbm, *scr):
            _sc_scatter_impl(True, idx_hbm, vals_hbm, out_hbm, *scr)
    else:
        @functools.partial(pl.kernel, out_type=out_type, mesh=mesh,
                           scratch_types=scratch, compiler_params=cp)
        def k(idx_hbm, out_hbm, *scr):
            _sc_scatter_impl(False, idx_hbm, None, out_hbm, *scr)
    return k


_sc_scatter_vals = _make_sc_scatter(True)
_sc_scatter_ones = _make_sc_scatter(False)


def _prep_body(x_ref, batch_ref, u_ref, wsrc_ref, wdst_ref, wu_ref, b1_ref,
               a_ref, b_ref):
    ue = u_ref[...] @ wu_ref[...] + b1_ref[...]
    oh = (batch_ref[...] ==
          lax.broadcasted_iota(jnp.int32, (N_BLK, G), 1)).astype(jnp.float32)
    a_ref[...] = (x_ref[...] @ wsrc_ref[...] + oh @ ue).astype(jnp.bfloat16)
    b_ref[...] = (x_ref[...] @ wdst_ref[...]).astype(jnp.bfloat16)


def _prep(x, batch2d, u, wsrc, wdst, wu, b1):
    nb = N // N_BLK
    return pl.pallas_call(
        _prep_body,
        grid=(nb,),
        in_specs=[
            pl.BlockSpec((N_BLK, FX), lambda i: (i, 0)),
            pl.BlockSpec((N_BLK, 1), lambda i: (i, 0)),
            _full((G, FU)), _full((FX, H)), _full((FX, H)),
            _full((FU, H)), _full((1, H)),
        ],
        out_specs=[
            pl.BlockSpec((N_BLK, H), lambda i: (i, 0)),
            pl.BlockSpec((N_BLK, H), lambda i: (i, 0)),
        ],
        out_shape=[
            jax.ShapeDtypeStruct((N, H), jnp.bfloat16),
            jax.ShapeDtypeStruct((N, H), jnp.bfloat16),
        ],
    )(x, batch2d, u, wsrc, wdst, wu, b1)


def _edge_body(gab_ref, e_ref, we_ref, w2_ref, b2_ref, out_ref):
    g = (gab_ref[:, :H].astype(jnp.float32)
         + gab_ref[:, H:].astype(jnp.float32))
    h = jnp.maximum(g + e_ref[...] @ we_ref[...], 0.0)
    out_ref[...] = h @ w2_ref[...] + b2_ref[...]


def _edge(gab, e, we, w2, b2):
    nb = E // E_BLK
    return pl.pallas_call(
        _edge_body,
        grid=(nb,),
        in_specs=[
            pl.BlockSpec((E_BLK, 2 * H), lambda i: (i, 0)),
            pl.BlockSpec((E_BLK, FE), lambda i: (i, 0)),
            _full((FE, H)), _full((H, FE)), _full((1, FE)),
        ],
        out_specs=pl.BlockSpec((E_BLK, FE), lambda i: (i, 0)),
        out_shape=jax.ShapeDtypeStruct((E, FE), jnp.float32),
    )(gab, e, we, w2, b2)


def _node_body(x_ref, s0_ref, s1_ref, c0_ref, c1c_ref, batch_ref, u_ref,
               vx_ref, va_ref, vu_ref, c1_ref, v2_ref, c2_ref,
               gwu_ref, gwx_ref, g1_ref, g2_ref, g2b_ref,
               xnew_ref, unew_ref, scr_ref):
    pid = pl.program_id(0)
    nb = pl.num_programs(0)

    @pl.when(pid == 0)
    def _():
        scr_ref[...] = jnp.zeros_like(scr_ref)

    agg = ((s0_ref[...] + s1_ref[...])
           / jnp.maximum(c0_ref[...] + c1c_ref[...], 1.0))
    uu = u_ref[...] @ vu_ref[...] + c1_ref[...]
    oh = (batch_ref[...] ==
          lax.broadcasted_iota(jnp.int32, (N_BLK, G), 1)).astype(jnp.float32)
    hx = jnp.maximum(x_ref[...] @ vx_ref[...] + agg @ va_ref[...] + oh @ uu,
                     0.0)
    xn = hx @ v2_ref[...] + c2_ref[...]
    xnew_ref[...] = xn
    ones = jnp.ones((N_BLK, FX), jnp.float32)
    scr_ref[...] += oh.T @ jnp.concatenate([xn, ones], axis=1)

    @pl.when(pid == nb - 1)
    def _():
        xagg = scr_ref[:, :FX] / jnp.maximum(scr_ref[:, FX:], 1.0)
        gin = jnp.maximum(
            u_ref[...] @ gwu_ref[...] + xagg @ gwx_ref[...] + g1_ref[...], 0.0)
        unew_ref[...] = gin @ g2_ref[...] + g2b_ref[...]


def _node(x, s0, s1, c0, c1c, batch2d, u, vx, va, vu, c1, v2, c2, gwu, gwx,
          g1, g2, g2b):
    nb = N // N_BLK
    return pl.pallas_call(
        _node_body,
        grid=(nb,),
        in_specs=[
            pl.BlockSpec((N_BLK, FX), lambda i: (i, 0)),
            pl.BlockSpec((N_BLK, FE), lambda i: (i, 0)),
            pl.BlockSpec((N_BLK, FE), lambda i: (i, 0)),
            pl.BlockSpec((N_BLK, FE), lambda i: (i, 0)),
            pl.BlockSpec((N_BLK, FE), lambda i: (i, 0)),
            pl.BlockSpec((N_BLK, 1), lambda i: (i, 0)),
            _full((G, FU)),
            _full((FX, H)), _full((FE, H)), _full((FU, H)), _full((1, H)),
            _full((H, FX)), _full((1, FX)),
            _full((FU, H)), _full((FX, H)), _full((1, H)), _full((H, FU)),
            _full((1, FU)),
        ],
        out_specs=[
            pl.BlockSpec((N_BLK, FX), lambda i: (i, 0)),
            _full((G, FU)),
        ],
        out_shape=[
            jax.ShapeDtypeStruct((N, FX), jnp.float32),
            jax.ShapeDtypeStruct((G, FU), jnp.float32),
        ],
        scratch_shapes=[pltpu.VMEM((G, 2 * FX), jnp.float32)],
    )(x, s0, s1, c0, c1c, batch2d, u, vx, va, vu, c1, v2, c2, gwu, gwx, g1,
      g2, g2b)


def _out_body(u1_ref, u2_ref, ow1_ref, ow2_ref, o1_ref, o2_ref, o2b_ref,
              out_ref):
    h = jnp.maximum(
        u1_ref[...] @ ow1_ref[...] + u2_ref[...] @ ow2_ref[...] + o1_ref[...],
        0.0)
    out_ref[...] = h @ o2_ref[...] + o2b_ref[...]


def _out(u1s, u2s, ow1, ow2, o1, o2, o2b):
    k = u1s.shape[0]
    return pl.pallas_call(
        _out_body,
        grid=(1,),
        in_specs=[_full((k, FU)), _full((k, FU)), _full((FU, H)),
                  _full((FU, H)), _full((1, H)), _full((H, 2)), _full((1, 2))],
        out_specs=_full((k, 2)),
        out_shape=jax.ShapeDtypeStruct((k, 2), jnp.float32),
    )(u1s, u2s, ow1, ow2, o1, o2, o2b)


def _split_gnn_params(p):
    w1, b1, w2, b2 = p['edge']
    ew = dict(wsrc=w1[:FX], wdst=w1[FX:2 * FX], we=w1[2 * FX:2 * FX + FE],
              wu=w1[2 * FX + FE:], b1=b1.reshape(1, H), w2=w2,
              b2=b2.reshape(1, FE))
    v1, c1, v2, c2 = p['node']
    nw = dict(vx=v1[:FX], va=v1[FX:FX + FE], vu=v1[FX + FE:],
              c1=c1.reshape(1, H), v2=v2, c2=c2.reshape(1, FX))
    g1w, g1b, g2w, g2b = p['glob']
    gw = dict(gwu=g1w[:FU], gwx=g1w[FU:], g1=g1b.reshape(1, H), g2=g2w,
              g2b=g2b.reshape(1, FU))
    return ew, nw, gw


class _GnnState:
    def __init__(self, p, x, ei, e, u, batch):
        self.ew, self.nw, self.gw = _split_gnn_params(p)
        self.src, self.dst = ei[0], ei[1]
        self.batch2d = batch.reshape(N, 1)
        self.x, self.e, self.u = x, e, u
        self.cparts = None
        self.us = []


def _step_prep(st):
    ew = st.ew
    return _prep(st.x, st.batch2d, st.u, ew['wsrc'], ew['wdst'], ew['wu'],
                 ew['b1'])


def _step_node(st, sparts):
    nw, gw = st.nw, st.gw
    st.x, u_new = _node(st.x, sparts[0], sparts[1], st.cparts[0],
                        st.cparts[1], st.batch2d, st.u, nw['vx'], nw['va'],
                        nw['vu'], nw['c1'], nw['v2'], nw['c2'], gw['gwu'],
                        gw['gwx'], gw['g1'], gw['g2'], gw['g2b'])
    st.u = u_new
    st.us.append(u_new)


def kernel(x1, edge_index1, e1, u1, batch1, x2, edge_index2, e2, u2, batch2,
           params):
    n_iters = 2
    s1 = _GnnState(params['gnn1'], x1, edge_index1, e1, u1, batch1)
    s2 = _GnnState(params['gnn2'], x2, edge_index2, e2, u2, batch2)
    s1.cparts = _sc_scatter_ones(s1.dst)
    s2.cparts = _sc_scatter_ones(s2.dst)
    for it in range(n_iters):
        for st in (s1, s2):
            a, b = _step_prep(st)
            gab = _sc_gather(a, b, st.src, st.dst)
            en = _edge(gab, st.e, st.ew['we'], st.ew['w2'], st.ew['b2'])
            sp = _sc_scatter_vals(st.dst, en)
            st.e = en
            _step_node(st, sp)
    u1s = jnp.stack(s1.us)
    u2s = jnp.stack(s2.us)
    ow, o1, o2, o2b = params['out']
    outs = _out(u1s.reshape(n_iters * G, FU), u2s.reshape(n_iters * G, FU),
                ow[:FU], ow[FU:], o1.reshape(1, H), o2, o2b.reshape(1, 2))
    return outs.reshape(n_iters, G, 2)
